# baseline (speedup 1.0000x reference)
import functools

import jax
import jax.numpy as jnp
from jax import lax
from jax.experimental import pallas as pl
from jax.experimental.pallas import tpu as pltpu
from jax.experimental.pallas import tpu_sc as plsc

_LEN = 200
_D = 1024
_B = 1024 * 50
_LPAD = 208
_TBL = 256
_WPAD = 256

_info = plsc.get_sparse_core_info()
_NC, _NS = _info.num_cores, _info.num_subcores
_NW = _NC * _NS
_BPW = _B // _NW

_mesh = plsc.VectorSubcoreMesh(core_axis_name="c", subcore_axis_name="s")

_TB = 2048
_G = _B // _TB


def _make_match_ids(n):
    bpw = n // _NW

    @functools.partial(
        pl.kernel,
        mesh=_mesh,
        compiler_params=pltpu.CompilerParams(needs_layout_passes=False),
        out_type=jax.ShapeDtypeStruct((n,), jnp.int32),
        scratch_types=[
            pltpu.VMEM((bpw,), jnp.int32),
            pltpu.VMEM((_LPAD,), jnp.int32),
            pltpu.VMEM((_TBL,), jnp.int32),
        ],
    )
    def match_ids(ids_hbm, iid_hbm, out_hbm, idsv, iidv, posv):
        cid = lax.axis_index("c")
        sid = lax.axis_index("s")
        wid = sid * _NC + cid
        base = wid * bpw

        pltpu.sync_copy(ids_hbm.at[pl.ds(base, bpw)], idsv)
        pltpu.sync_copy(iid_hbm, iidv)

        zeros = jnp.zeros((16,), jnp.int32)
        for k in range(_TBL // 16):
            posv[pl.ds(k * 16, 16)] = zeros
        lanes = lax.iota(jnp.int32, 16)
        for jb in reversed(range(_LPAD // 16)):
            vals = iidv[pl.ds(jb * 16, 16)]
            plsc.store_scatter(posv, [vals], jb * 16 + lanes)

        def map_body(k, carry):
            t = idsv[pl.ds(k * 16, 16)]
            idsv[pl.ds(k * 16, 16)] = plsc.load_gather(posv, [t])
            return carry

        lax.fori_loop(0, bpw // 16, map_body, 0)

        pltpu.sync_copy(idsv, out_hbm.at[pl.ds(base, bpw)])

    return match_ids


_G0 = (_G + 1) // 2
_G1 = _G - _G0
_N0 = _G0 * _TB
_N1 = _G1 * _TB

_match0 = _make_match_ids(_N0)
_match1 = _make_match_ids(_N1)


def _compare_body(idx_ref, w_ref, out_ref):
    ids = idx_ref[0, 0, :]
    cols = lax.broadcasted_iota(jnp.int32, (_TB, _WPAD), 1)
    onehot = (ids[:, None] == cols).astype(jnp.bfloat16)
    out_ref[...] = jnp.dot(onehot, w_ref[...],
                           preferred_element_type=jnp.float32)


def _alias_body(prev_ref, idx_ref, w_ref, out_ref):
    del prev_ref
    _compare_body(idx_ref, w_ref, out_ref)


_materialize0 = pl.pallas_call(
    _compare_body,
    grid=(_G0,),
    in_specs=[
        pl.BlockSpec((1, 1, _TB), lambda i: (i, 0, 0)),
        pl.BlockSpec((_WPAD, _D), lambda i: (0, 0)),
    ],
    out_specs=pl.BlockSpec((_TB, _D), lambda i: (i, 0)),
    out_shape=jax.ShapeDtypeStruct((_B, _D), jnp.float32),
)

_materialize1 = pl.pallas_call(
    _alias_body,
    grid=(_G1,),
    in_specs=[
        pl.BlockSpec(memory_space=pl.ANY),
        pl.BlockSpec((1, 1, _TB), lambda i: (i, 0, 0)),
        pl.BlockSpec((_WPAD, _D), lambda i: (0, 0)),
    ],
    out_specs=pl.BlockSpec((_TB, _D), lambda i: (i + _G0, 0)),
    out_shape=jax.ShapeDtypeStruct((_B, _D), jnp.float32),
    input_output_aliases={0: 0},
)


def kernel(prompt_token_ids, input_ids, W):
    ids = prompt_token_ids.reshape(-1).astype(jnp.int32)
    pad = jnp.arange(_LEN, _LPAD, dtype=jnp.int32) + (_TBL - _LPAD)
    iid = jnp.concatenate([input_ids.astype(jnp.int32), pad])
    idx0 = _match0(ids[:_N0], iid)
    idx1 = _match1(ids[_N0:], iid)
    wb = jnp.zeros((_WPAD, _D), jnp.bfloat16).at[:_LEN].set(
        W.astype(jnp.bfloat16))
    out = _materialize0(idx0.reshape(_G0, 1, _TB), wb)
    return _materialize1(out, idx1.reshape(_G1, 1, _TB), wb)


# speedup vs baseline: 1.0497x; 1.0497x over previous
import functools

import jax
import jax.numpy as jnp
from jax import lax
from jax.experimental import pallas as pl
from jax.experimental.pallas import tpu as pltpu
from jax.experimental.pallas import tpu_sc as plsc

_LEN = 200
_D = 1024
_B = 1024 * 50
_LPAD = 208
_TBL = 256
_WPAD = 256

_info = plsc.get_sparse_core_info()
_NC, _NS = _info.num_cores, _info.num_subcores
_NW = _NC * _NS
_BPW = _B // _NW

_mesh = plsc.VectorSubcoreMesh(core_axis_name="c", subcore_axis_name="s")

_TB = 2048
_G = _B // _TB


def _make_match_ids(n):
    bpw = n // _NW

    @functools.partial(
        pl.kernel,
        mesh=_mesh,
        compiler_params=pltpu.CompilerParams(needs_layout_passes=False),
        out_type=jax.ShapeDtypeStruct((n,), jnp.int32),
        scratch_types=[
            pltpu.VMEM((bpw,), jnp.int32),
            pltpu.VMEM((_LPAD,), jnp.int32),
            pltpu.VMEM((_TBL,), jnp.int32),
        ],
    )
    def match_ids(ids_hbm, iid_hbm, out_hbm, idsv, iidv, posv):
        cid = lax.axis_index("c")
        sid = lax.axis_index("s")
        wid = sid * _NC + cid
        base = wid * bpw

        pltpu.sync_copy(ids_hbm.at[pl.ds(base, bpw)], idsv)
        pltpu.sync_copy(iid_hbm, iidv)

        zeros = jnp.zeros((16,), jnp.int32)
        for k in range(_TBL // 16):
            posv[pl.ds(k * 16, 16)] = zeros
        lanes = lax.iota(jnp.int32, 16)
        for jb in reversed(range(_LPAD // 16)):
            vals = iidv[pl.ds(jb * 16, 16)]
            plsc.store_scatter(posv, [vals], jb * 16 + lanes)

        def map_body(k, carry):
            t = idsv[pl.ds(k * 16, 16)]
            idsv[pl.ds(k * 16, 16)] = plsc.load_gather(posv, [t])
            return carry

        lax.fori_loop(0, bpw // 16, map_body, 0)

        pltpu.sync_copy(idsv, out_hbm.at[pl.ds(base, bpw)])

    return match_ids


_match_ids = _make_match_ids(_B)


def _compare_body(idx_ref, w_ref, out_ref):
    ids = idx_ref[0, 0, :]
    cols = lax.broadcasted_iota(jnp.int32, (_TB, _WPAD), 1)
    onehot = (ids[:, None] == cols).astype(jnp.bfloat16)
    out_ref[...] = jnp.dot(onehot, w_ref[...],
                           preferred_element_type=jnp.float32)


_materialize = pl.pallas_call(
    _compare_body,
    grid=(_G,),
    in_specs=[
        pl.BlockSpec((1, 1, _TB), lambda i: (i, 0, 0)),
        pl.BlockSpec((_WPAD, _D), lambda i: (0, 0)),
    ],
    out_specs=pl.BlockSpec((_TB, _D), lambda i: (i, 0)),
    out_shape=jax.ShapeDtypeStruct((_B, _D), jnp.float32),
)


def kernel(prompt_token_ids, input_ids, W):
    ids = prompt_token_ids.reshape(-1).astype(jnp.int32)
    pad = jnp.arange(_LEN, _LPAD, dtype=jnp.int32) + (_TBL - _LPAD)
    iid = jnp.concatenate([input_ids.astype(jnp.int32), pad])
    idx = _match_ids(ids, iid)
    wb = jnp.zeros((_WPAD, _D), jnp.bfloat16).at[:_LEN].set(
        W.astype(jnp.bfloat16))
    return _materialize(idx.reshape(_G, 1, _TB), wb)


# speedup vs baseline: 1.0529x; 1.0031x over previous
import functools

import jax
import jax.numpy as jnp
from jax import lax
from jax.experimental import pallas as pl
from jax.experimental.pallas import tpu as pltpu
from jax.experimental.pallas import tpu_sc as plsc

_LEN = 200
_D = 1024
_B = 1024 * 50
_LPAD = 208
_TBL = 256
_WPAD = 256

_info = plsc.get_sparse_core_info()
_NC, _NS = _info.num_cores, _info.num_subcores
_NW = _NC * _NS
_BPW = _B // _NW

_mesh = plsc.VectorSubcoreMesh(core_axis_name="c", subcore_axis_name="s")

_TB = 2048
_G = _B // _TB


def _make_match_ids(n):
    bpw = n // _NW

    @functools.partial(
        pl.kernel,
        mesh=_mesh,
        compiler_params=pltpu.CompilerParams(needs_layout_passes=False),
        out_type=jax.ShapeDtypeStruct((n,), jnp.int32),
        scratch_types=[
            pltpu.VMEM((bpw,), jnp.int32),
            pltpu.VMEM((_LPAD,), jnp.int32),
            pltpu.VMEM((_TBL,), jnp.int32),
            pltpu.SemaphoreType.DMA,
        ],
    )
    def match_ids(ids_hbm, iid_hbm, out_hbm, idsv, iidv, posv, isem):
        cid = lax.axis_index("c")
        sid = lax.axis_index("s")
        wid = sid * _NC + cid
        base = wid * bpw

        icopy = pltpu.async_copy(ids_hbm.at[pl.ds(base, bpw)], idsv, isem)
        pltpu.sync_copy(iid_hbm, iidv)

        zeros = jnp.zeros((16,), jnp.int32)
        for k in range(_TBL // 16):
            posv[pl.ds(k * 16, 16)] = zeros
        lanes = lax.iota(jnp.int32, 16)
        for jb in reversed(range(_LPAD // 16)):
            vals = iidv[pl.ds(jb * 16, 16)]
            plsc.store_scatter(posv, [vals], jb * 16 + lanes)

        icopy.wait()

        def map_body(k, carry):
            t = idsv[pl.ds(k * 16, 16)]
            idsv[pl.ds(k * 16, 16)] = plsc.load_gather(posv, [t])
            return carry

        lax.fori_loop(0, bpw // 16, map_body, 0)

        pltpu.sync_copy(idsv, out_hbm.at[pl.ds(base, bpw)])

    return match_ids


_match_ids = _make_match_ids(_B)


def _compare_body(idx_ref, w_ref, out_ref):
    ids = idx_ref[0, 0, :]
    cols = lax.broadcasted_iota(jnp.int32, (_TB, _WPAD), 1)
    onehot = (ids[:, None] == cols).astype(jnp.bfloat16)
    out_ref[...] = jnp.dot(onehot, w_ref[...],
                           preferred_element_type=jnp.float32)


_materialize = pl.pallas_call(
    _compare_body,
    grid=(_G,),
    in_specs=[
        pl.BlockSpec((1, 1, _TB), lambda i: (i, 0, 0)),
        pl.BlockSpec((_WPAD, _D), lambda i: (0, 0)),
    ],
    out_specs=pl.BlockSpec((_TB, _D), lambda i: (i, 0)),
    out_shape=jax.ShapeDtypeStruct((_B, _D), jnp.float32),
)


def kernel(prompt_token_ids, input_ids, W):
    ids = prompt_token_ids.reshape(-1).astype(jnp.int32)
    pad = jnp.arange(_LEN, _LPAD, dtype=jnp.int32) + (_TBL - _LPAD)
    iid = jnp.concatenate([input_ids.astype(jnp.int32), pad])
    idx = _match_ids(ids, iid)
    wb = jnp.zeros((_WPAD, _D), jnp.bfloat16).at[:_LEN].set(
        W.astype(jnp.bfloat16))
    return _materialize(idx.reshape(_G, 1, _TB), wb)
